# baseline (speedup 1.0000x reference)
import functools

import jax
import jax.numpy as jnp
from jax import lax
from jax.experimental import pallas as pl
from jax.experimental.pallas import tpu as pltpu
from jax.experimental.pallas import tpu_sc as plsc

_VOCAB = 1000000
_HIDDEN = 64
_BATCH = 16384

_info = plsc.get_sparse_core_info()
_NC, _NS = _info.num_cores, _info.num_subcores
_NW = _NC * _NS
_BPW = _BATCH // _NW
_CHUNK = 128
_NCHUNK = _BPW // _CHUNK

_mesh = plsc.VectorSubcoreMesh(core_axis_name="c", subcore_axis_name="s")


@functools.partial(
    pl.kernel,
    mesh=_mesh,
    out_type=jax.ShapeDtypeStruct((_BATCH, _HIDDEN), jnp.float32),
    scratch_types=[
        pltpu.VMEM((_BPW,), jnp.int32),
        pltpu.VMEM((_BPW, _HIDDEN), jnp.float32),
        pltpu.SemaphoreType.DMA,
    ],
    compiler_params=pltpu.CompilerParams(use_tc_tiling_on_sc=False),
)
def _gather_kernel(src_hbm, emb_hbm, out_hbm, idx_v, rows_v, sem):
    wid = lax.axis_index("s") * _NC + lax.axis_index("c")
    base = wid * _BPW
    pltpu.sync_copy(src_hbm.at[pl.ds(base, _BPW)], idx_v)
    copies = [
        pltpu.async_copy(
            emb_hbm.at[idx_v.at[pl.ds(j * _CHUNK, _CHUNK)]],
            rows_v.at[pl.ds(j * _CHUNK, _CHUNK)],
            sem,
        )
        for j in range(_NCHUNK)
    ]
    for cp in copies:
        cp.wait()
    pltpu.sync_copy(rows_v, out_hbm.at[pl.ds(base, _BPW)])


def kernel(source, hidden, cell, emb):
    del hidden, cell
    return _gather_kernel(source, emb)


# speedup vs baseline: 1.1225x; 1.1225x over previous
import functools

import jax
import jax.numpy as jnp
from jax import lax
from jax.experimental import pallas as pl
from jax.experimental.pallas import tpu as pltpu
from jax.experimental.pallas import tpu_sc as plsc

_VOCAB = 1000000
_HIDDEN = 64
_PADH = 128
_BATCH = 16384

_info = plsc.get_sparse_core_info()
_NC, _NS = _info.num_cores, _info.num_subcores
_NW = _NC * _NS
_BPW = _BATCH // _NW
_CHUNK = 128
_NCHUNK = _BPW // _CHUNK

_mesh = plsc.VectorSubcoreMesh(core_axis_name="c", subcore_axis_name="s")


@functools.partial(
    pl.kernel,
    mesh=_mesh,
    out_type=jax.ShapeDtypeStruct((_BATCH, _PADH), jnp.float32),
    scratch_types=[
        pltpu.VMEM((_BPW,), jnp.int32),
        pltpu.VMEM((_BPW, _PADH), jnp.float32),
        pltpu.SemaphoreType.DMA,
    ],
)
def _gather_kernel(src_hbm, emb_hbm, out_hbm, idx_v, rows_v, sem):
    wid = lax.axis_index("s") * _NC + lax.axis_index("c")
    base = wid * _BPW
    pltpu.sync_copy(src_hbm.at[pl.ds(base, _BPW)], idx_v)
    copies = [
        pltpu.async_copy(
            emb_hbm.at[idx_v.at[pl.ds(j * _CHUNK, _CHUNK)]],
            rows_v.at[pl.ds(j * _CHUNK, _CHUNK)],
            sem,
        )
        for j in range(_NCHUNK)
    ]
    for cp in copies:
        cp.wait()
    pltpu.sync_copy(rows_v, out_hbm.at[pl.ds(base, _BPW)])


def kernel(source, hidden, cell, emb):
    del hidden, cell
    embp = jnp.pad(emb, ((0, 0), (0, _PADH - _HIDDEN)))
    return _gather_kernel(source, embp)[:, :_HIDDEN]


# speedup vs baseline: 3.2160x; 2.8651x over previous
import functools

import jax
import jax.numpy as jnp
from jax import lax
from jax.experimental import pallas as pl
from jax.experimental.pallas import tpu as pltpu
from jax.experimental.pallas import tpu_sc as plsc

_VOCAB = 1000000
_HIDDEN = 64
_BATCH = 16384

_info = plsc.get_sparse_core_info()
_NC, _NS = _info.num_cores, _info.num_subcores
_NW = _NC * _NS
_NG = 8
_NB = _NW // _NG
_BPB = _BATCH // _NB
_L = 16
_TB = 32
_NBATCH = _BPB // _TB

_mesh = plsc.VectorSubcoreMesh(core_axis_name="c", subcore_axis_name="s")


@functools.partial(
    pl.kernel,
    mesh=_mesh,
    out_type=jax.ShapeDtypeStruct((_HIDDEN, _BATCH), jnp.float32),
    scratch_types=[
        pltpu.VMEM((_BPB,), jnp.int32),
        pltpu.VMEM((2, _TB * 8, 128), jnp.float32),
        *[pltpu.VMEM((_BPB,), jnp.float32) for _ in range(_NG)],
        pltpu.SemaphoreType.DMA,
    ],
    compiler_params=pltpu.CompilerParams(
        needs_layout_passes=False, disable_bounds_checks=True
    ),
)
def _gather_kernel(src_hbm, embt_hbm, outt_hbm, idx_v, tiles_v, *rest):
    rows = rest[:_NG]
    sem = rest[_NG]
    wid = lax.axis_index("s") * _NC + lax.axis_index("c")
    g = wid // _NB
    ib = wid % _NB
    f0 = g * _NG
    pltpu.sync_copy(src_hbm.at[pl.ds(ib * _BPB, _BPB)], idx_v)

    lanes = lax.iota(jnp.int32, _L)

    def fire(b):
        slot = b & 1
        for h in range(2):
            vec = idx_v[pl.ds(b * _TB + h * _L, _L)]
            for jj in range(_L):
                r = jnp.max(jnp.where(lanes == jj, vec, 0))
                c0 = pl.multiple_of((r >> 7) * 128, 128)
                t = h * _L + jj
                pltpu.async_copy(
                    embt_hbm.at[pl.ds(f0, _NG), pl.ds(c0, 128)],
                    tiles_v.at[slot, pl.ds(t * 8, 8), :],
                    sem,
                )

    def drain(b):
        slot = b & 1
        for q in range(4):
            pltpu.make_async_copy(
                embt_hbm.at[pl.ds(0, 64), pl.ds(0, 128)],
                tiles_v.at[slot, pl.ds(q * 64, 64), :],
                sem,
            ).wait()

    def extract(b):
        slot = b & 1
        slot_v = jnp.full((_L,), slot, jnp.int32)
        for h in range(2):
            vec = idx_v[pl.ds(b * _TB + h * _L, _L)]
            lane_v = vec & 127
            col0 = b * _TB + h * _L
            for k in range(_NG):
                row_v = lanes * 8 + (128 * h + k)
                v = plsc.load_gather(tiles_v, [slot_v, row_v, lane_v])
                rows[k][pl.ds(col0, _L)] = v

    fire(jnp.int32(0))

    def body(i, carry):
        fire(i + 1)
        drain(i)
        extract(i)
        return carry

    lax.fori_loop(0, _NBATCH - 1, body, 0)
    last = jnp.int32(_NBATCH - 1)
    drain(last)
    extract(last)

    for k in range(_NG):
        pltpu.sync_copy(
            rows[k], outt_hbm.at[f0 + k].at[pl.ds(ib * _BPB, _BPB)]
        )


def kernel(source, hidden, cell, emb):
    del hidden, cell
    outt = _gather_kernel(source, emb.T)
    return outt.T
